# baseline (speedup 1.0000x reference)
import functools

import jax
import jax.numpy as jnp
from jax import lax
from jax.experimental import pallas as pl
from jax.experimental.pallas import tpu as pltpu
from jax.experimental.pallas import tpu_sc as plsc

N = 10000
E = 320000
F_IN = 128
HID = 128
OUT = 64

NC = 2
NS = 16
NW = NC * NS

C = 128
EW = E // NW
FULL = EW // C
TAIL = EW - FULL * C

NACC = 10112
RPT = NACC // NS
RTAIL = RPT - (RPT // C) * C

_MESH = plsc.VectorSubcoreMesh(
    core_axis_name="c", subcore_axis_name="s", num_cores=NC, num_subcores=NS)


def _zero_vmem_1d(ref, n):
    def body(i, carry):
        ref[pl.ds(i * 16, 16)] = jnp.zeros((16,), jnp.float32)
        return carry
    lax.fori_loop(0, n // 16, body, 0)


def _deg_body(dst_hbm, out_hbm, didx0_v, didx1_v, didxt_v, ones_v, zeros_v,
              acc_sh, d0, d1, a0, a1):
    c = lax.axis_index("c")
    s = lax.axis_index("s")
    wid = c * NS + s
    rbase = pl.multiple_of(s * RPT, 8)
    ebase = pl.multiple_of(wid * EW, 8)

    def fill_ones(i, carry):
        ones_v[pl.ds(i * 16, 16)] = jnp.full((16,), 1.0, jnp.float32)
        return carry
    lax.fori_loop(0, C // 16, fill_ones, 0)
    _zero_vmem_1d(zeros_v, RPT)
    pltpu.sync_copy(zeros_v, acc_sh.at[pl.ds(rbase, RPT)])
    plsc.subcore_barrier()

    def start_didx(i, didx, sem):
        pltpu.async_copy(dst_hbm.at[pl.ds(ebase + i * C, C)], didx, sem)

    def wait_didx(i, didx, sem):
        pltpu.make_async_copy(dst_hbm.at[pl.ds(ebase + i * C, C)], didx,
                              sem).wait()

    def start_add(didx, sem):
        pltpu.async_copy(ones_v, acc_sh.at[didx], sem, add=True)

    def wait_add(didx, sem):
        pltpu.make_async_copy(ones_v, acc_sh.at[didx], sem).wait()

    start_didx(0, didx0_v, d0)
    start_didx(1, didx1_v, d1)

    def chunk(j, carry):
        i0 = j * 2
        wait_didx(i0, didx0_v, d0)
        start_add(didx0_v, a0)
        wait_didx(i0 + 1, didx1_v, d1)
        start_add(didx1_v, a1)
        wait_add(didx0_v, a0)
        start_didx(i0 + 2, didx0_v, d0)
        wait_add(didx1_v, a1)
        start_didx(i0 + 3, didx1_v, d1)
        return carry
    lax.fori_loop(0, FULL // 2 - 1, chunk, 0)
    last = FULL - 2
    wait_didx(last, didx0_v, d0)
    start_add(didx0_v, a0)
    wait_didx(last + 1, didx1_v, d1)
    start_add(didx1_v, a1)
    wait_add(didx0_v, a0)
    wait_add(didx1_v, a1)
    pltpu.sync_copy(dst_hbm.at[pl.ds(ebase + FULL * C, TAIL)], didxt_v)
    pltpu.sync_copy(ones_v.at[pl.ds(0, TAIL)], acc_sh.at[didxt_v], add=True)
    plsc.subcore_barrier()
    obase = pl.multiple_of(c * NACC + s * RPT, 8)
    pltpu.sync_copy(acc_sh.at[pl.ds(rbase, RPT)], zeros_v)
    pltpu.sync_copy(zeros_v, out_hbm.at[pl.ds(obase, RPT)])


_deg_call = pl.kernel(
    _deg_body,
    out_type=jax.ShapeDtypeStruct((NC * NACC,), jnp.float32),
    mesh=_MESH,
    scratch_types=[
        pltpu.VMEM((C,), jnp.int32),
        pltpu.VMEM((C,), jnp.int32),
        pltpu.VMEM((TAIL,), jnp.int32),
        pltpu.VMEM((C,), jnp.float32),
        pltpu.VMEM((RPT,), jnp.float32),
        pltpu.VMEM_SHARED((NACC,), jnp.float32),
        pltpu.SemaphoreType.DMA,
        pltpu.SemaphoreType.DMA,
        pltpu.SemaphoreType.DMA,
        pltpu.SemaphoreType.DMA,
    ],
)


def _agg_body(hs_hbm, src_hbm, dst_hbm, out_hbm,
              sidx_v, didx0_v, didx1_v, didxt_v, rows0_v, rows1_v, acc_sh,
              g0, g1, s0, s1, d0, d1, d):
    c = lax.axis_index("c")
    s = lax.axis_index("s")
    wid = c * NS + s
    rbase = pl.multiple_of(s * RPT, 8)
    ebase = pl.multiple_of(wid * EW, 8)

    pltpu.sync_copy(src_hbm.at[pl.ds(ebase, EW)], sidx_v)

    def zrow(i, carry):
        for j in range(d // 16):
            rows0_v[i, pl.ds(j * 16, 16)] = jnp.zeros((16,), jnp.float32)
        return carry
    lax.fori_loop(0, C, zrow, 0)

    def zcopy(i, carry):
        pltpu.sync_copy(rows0_v, acc_sh.at[pl.ds(rbase + i * C, C)])
        return carry
    lax.fori_loop(0, RPT // C, zcopy, 0)
    pltpu.sync_copy(rows0_v.at[pl.ds(0, RTAIL)],
                    acc_sh.at[pl.ds(rbase + (RPT // C) * C, RTAIL)])
    plsc.subcore_barrier()

    def start_gather(i, rows, sem):
        pltpu.async_copy(hs_hbm.at[sidx_v.at[pl.ds(i * C, C)]], rows, sem)

    def wait_gather(i, rows, sem):
        pltpu.make_async_copy(hs_hbm.at[sidx_v.at[pl.ds(i * C, C)]], rows,
                              sem).wait()

    def start_didx(i, didx, sem):
        pltpu.async_copy(dst_hbm.at[pl.ds(ebase + i * C, C)], didx, sem)

    def wait_didx(i, didx, sem):
        pltpu.make_async_copy(dst_hbm.at[pl.ds(ebase + i * C, C)], didx,
                              sem).wait()

    def start_scat(rows, didx, sem):
        pltpu.async_copy(rows, acc_sh.at[didx], sem, add=True)

    def wait_scat(rows, didx, sem):
        pltpu.make_async_copy(rows, acc_sh.at[didx], sem).wait()

    start_didx(0, didx0_v, d0)
    start_didx(1, didx1_v, d1)
    start_gather(0, rows0_v, g0)
    start_gather(1, rows1_v, g1)

    def chunk(j, carry):
        i0 = j * 2
        wait_gather(i0, rows0_v, g0)
        wait_didx(i0, didx0_v, d0)
        start_scat(rows0_v, didx0_v, s0)
        wait_gather(i0 + 1, rows1_v, g1)
        wait_didx(i0 + 1, didx1_v, d1)
        start_scat(rows1_v, didx1_v, s1)
        wait_scat(rows0_v, didx0_v, s0)
        start_gather(i0 + 2, rows0_v, g0)
        start_didx(i0 + 2, didx0_v, d0)
        wait_scat(rows1_v, didx1_v, s1)
        start_gather(i0 + 3, rows1_v, g1)
        start_didx(i0 + 3, didx1_v, d1)
        return carry
    last = FULL - 2
    lax.fori_loop(0, FULL // 2 - 1, chunk, 0)
    wait_gather(last, rows0_v, g0)
    wait_didx(last, didx0_v, d0)
    start_scat(rows0_v, didx0_v, s0)
    wait_gather(last + 1, rows1_v, g1)
    wait_didx(last + 1, didx1_v, d1)
    start_scat(rows1_v, didx1_v, s1)
    wait_scat(rows0_v, didx0_v, s0)
    wait_scat(rows1_v, didx1_v, s1)
    pltpu.sync_copy(dst_hbm.at[pl.ds(ebase + FULL * C, TAIL)], didxt_v)
    pltpu.async_copy(hs_hbm.at[sidx_v.at[pl.ds(FULL * C, TAIL)]],
                     rows0_v.at[pl.ds(0, TAIL)], g0).wait()
    pltpu.sync_copy(rows0_v.at[pl.ds(0, TAIL)], acc_sh.at[didxt_v], add=True)
    plsc.subcore_barrier()

    def ocopy(i, carry):
        ob = pl.multiple_of(rbase + i * C, 8)
        pltpu.sync_copy(acc_sh.at[pl.ds(ob, C)], rows1_v)
        pltpu.sync_copy(rows1_v, out_hbm.at[c, pl.ds(ob, C)])
        return carry
    lax.fori_loop(0, RPT // C, ocopy, 0)
    ot = pl.multiple_of(rbase + (RPT // C) * C, 8)
    pltpu.sync_copy(acc_sh.at[pl.ds(ot, RTAIL)], rows1_v.at[pl.ds(0, RTAIL)])
    pltpu.sync_copy(rows1_v.at[pl.ds(0, RTAIL)], out_hbm.at[c, pl.ds(ot, RTAIL)])


def _make_agg(d, tc_tiling=True):
    return pl.kernel(
        functools.partial(_agg_body, d=d),
        out_type=jax.ShapeDtypeStruct((NC, NACC, d), jnp.float32),
        mesh=_MESH,
        compiler_params=pltpu.CompilerParams(use_tc_tiling_on_sc=tc_tiling),
        scratch_types=[
            pltpu.VMEM((EW,), jnp.int32),
            pltpu.VMEM((C,), jnp.int32),
            pltpu.VMEM((C,), jnp.int32),
            pltpu.VMEM((TAIL,), jnp.int32),
            pltpu.VMEM((C, d), jnp.float32),
            pltpu.VMEM((C, d), jnp.float32),
            pltpu.VMEM_SHARED((NACC, d), jnp.float32),
            pltpu.SemaphoreType.DMA,
            pltpu.SemaphoreType.DMA,
            pltpu.SemaphoreType.DMA,
            pltpu.SemaphoreType.DMA,
            pltpu.SemaphoreType.DMA,
            pltpu.SemaphoreType.DMA,
        ],
    )


_agg128 = _make_agg(HID)
_agg64 = _make_agg(OUT, tc_tiling=False)

BM = 400
NROW = NACC // 128


def _m1_body(x_ref, w_ref, dv_ref, hs_ref):
    h = jnp.dot(x_ref[...], w_ref[...], preferred_element_type=jnp.float32)
    hs_ref[...] = h * dv_ref[...]


_m1_call = pl.pallas_call(
    _m1_body,
    grid=(N // BM,),
    in_specs=[
        pl.BlockSpec((BM, F_IN), lambda i: (i, 0)),
        pl.BlockSpec((F_IN, HID), lambda i: (0, 0)),
        pl.BlockSpec((BM, HID), lambda i: (i, 0)),
    ],
    out_specs=pl.BlockSpec((BM, HID), lambda i: (i, 0)),
    out_shape=jax.ShapeDtypeStruct((N, HID), jnp.float32),
)


def _l2_body(a0_ref, a1_ref, hs1_ref, dv_ref, b1_ref, w2_ref, o_ref):
    act = (a0_ref[0] + a1_ref[0] + hs1_ref[...]) * dv_ref[...] + b1_ref[...]
    act = jnp.maximum(act, 0.0)
    h2 = jnp.dot(act, w2_ref[...], preferred_element_type=jnp.float32)
    o_ref[...] = h2 * dv_ref[:, :OUT]


_l2_call = pl.pallas_call(
    _l2_body,
    grid=(N // BM,),
    in_specs=[
        pl.BlockSpec((1, BM, HID), lambda i: (0, i, 0)),
        pl.BlockSpec((1, BM, HID), lambda i: (1, i, 0)),
        pl.BlockSpec((BM, HID), lambda i: (i, 0)),
        pl.BlockSpec((BM, HID), lambda i: (i, 0)),
        pl.BlockSpec((1, HID), lambda i: (0, 0)),
        pl.BlockSpec((HID, OUT), lambda i: (0, 0)),
    ],
    out_specs=pl.BlockSpec((BM, OUT), lambda i: (i, 0)),
    out_shape=jax.ShapeDtypeStruct((N, OUT), jnp.float32),
)


def _fin_body(a0_ref, a1_ref, hs2_ref, dv_ref, b2_ref, o_ref):
    t = a0_ref[0] + a1_ref[0] + hs2_ref[...]
    o = t * dv_ref[:, :OUT] + b2_ref[...]
    o_ref[...] = jax.nn.sigmoid(o)


_fin_call = pl.pallas_call(
    _fin_body,
    grid=(N // BM,),
    in_specs=[
        pl.BlockSpec((1, BM, OUT), lambda i: (0, i, 0)),
        pl.BlockSpec((1, BM, OUT), lambda i: (1, i, 0)),
        pl.BlockSpec((BM, OUT), lambda i: (i, 0)),
        pl.BlockSpec((BM, HID), lambda i: (i, 0)),
        pl.BlockSpec((1, OUT), lambda i: (0, 0)),
    ],
    out_specs=pl.BlockSpec((BM, OUT), lambda i: (i, 0)),
    out_shape=jax.ShapeDtypeStruct((N, OUT), jnp.float32),
)


def kernel(x, edge_index, W1, b1, W2, b2):
    src_p = edge_index[0]
    dst_f = edge_index[1]

    degf = _deg_call(dst_f)
    deg = degf[:N] + degf[NACC:NACC + N] + 1.0
    dinv = jnp.broadcast_to(lax.rsqrt(deg)[:, None], (N, HID))

    hs1 = _m1_call(x, W1, dinv)
    agg1 = _agg128(hs1, src_p, dst_f)
    hs2 = _l2_call(agg1, agg1, hs1, dinv, b1[None, :], W2)
    agg2 = _agg64(hs2, src_p, dst_f)
    return _fin_call(agg2, agg2, hs2, dinv, b2[None, :])


# speedup vs baseline: 1.1112x; 1.1112x over previous
import functools

import jax
import jax.numpy as jnp
from jax import lax
from jax.experimental import pallas as pl
from jax.experimental.pallas import tpu as pltpu
from jax.experimental.pallas import tpu_sc as plsc

N = 10000
E = 320000
F_IN = 128
HID = 128
OUT = 64

NC = 2
NS = 16
NW = NC * NS

C = 128
CHUNKS_PER_TILE = 80
EW = CHUNKS_PER_TILE * C
E_PAD = NW * EW
PAD = E_PAD - E

TRASH = 512
NACC = 10752
RPT = NACC // NS
ZB = 96

_MESH = plsc.VectorSubcoreMesh(
    core_axis_name="c", subcore_axis_name="s", num_cores=NC, num_subcores=NS)


def _zero_vmem_1d(ref, n):
    def body(i, carry):
        ref[pl.ds(i * 16, 16)] = jnp.zeros((16,), jnp.float32)
        return carry
    lax.fori_loop(0, n // 16, body, 0)


def _deg_body(dst_hbm, out_hbm, didx_v, ones_v, zeros_v, acc_sh, sem):
    del sem
    c = lax.axis_index("c")
    s = lax.axis_index("s")
    wid = c * NS + s
    rbase = pl.multiple_of(s * RPT, 8)

    pltpu.sync_copy(
        dst_hbm.at[pl.ds(pl.multiple_of(wid * CHUNKS_PER_TILE, 8),
                         CHUNKS_PER_TILE)], didx_v)

    def fill_ones(i, carry):
        ones_v[pl.ds(i * 16, 16)] = jnp.full((16,), 1.0, jnp.float32)
        return carry
    lax.fori_loop(0, C // 16, fill_ones, 0)
    _zero_vmem_1d(zeros_v, RPT)
    pltpu.sync_copy(zeros_v, acc_sh.at[pl.ds(rbase, RPT)])
    plsc.subcore_barrier()

    def chunk(i, carry):
        pltpu.sync_copy(ones_v, acc_sh.at[didx_v.at[i]], add=True)
        return carry
    lax.fori_loop(0, CHUNKS_PER_TILE, chunk, 0)
    plsc.subcore_barrier()
    obase = pl.multiple_of(c * NACC + s * RPT, 8)
    pltpu.sync_copy(acc_sh.at[pl.ds(rbase, RPT)], zeros_v)
    pltpu.sync_copy(zeros_v, out_hbm.at[pl.ds(obase, RPT)])


_deg_call = pl.kernel(
    _deg_body,
    out_type=jax.ShapeDtypeStruct((NC * NACC,), jnp.float32),
    mesh=_MESH,
    scratch_types=[
        pltpu.VMEM((CHUNKS_PER_TILE, C), jnp.int32),
        pltpu.VMEM((C,), jnp.float32),
        pltpu.VMEM((RPT,), jnp.float32),
        pltpu.VMEM_SHARED((NACC,), jnp.float32),
        pltpu.SemaphoreType.DMA,
    ],
)


def _agg_body(hs_hbm, src_hbm, dst_hbm, out_hbm,
              sidx_v, didx0_v, didx1_v, rows0_v, rows1_v, acc_sh,
              g0, g1, s0, s1, d0, d1, d):
    c = lax.axis_index("c")
    s = lax.axis_index("s")
    wid = c * NS + s
    rbase = pl.multiple_of(s * RPT, 8)
    cbase = pl.multiple_of(wid * CHUNKS_PER_TILE, 8)
    ebase = pl.multiple_of(wid * EW, 8)

    pltpu.sync_copy(src_hbm.at[pl.ds(cbase, CHUNKS_PER_TILE)], sidx_v)

    def zrow(i, carry):
        for j in range(d // 16):
            rows0_v[i, pl.ds(j * 16, 16)] = jnp.zeros((16,), jnp.float32)
        return carry
    lax.fori_loop(0, C, zrow, 0)

    def zcopy(i, carry):
        pltpu.sync_copy(rows0_v, acc_sh.at[pl.ds(rbase + i * C, C)])
        return carry
    lax.fori_loop(0, RPT // C, zcopy, 0)
    pltpu.sync_copy(rows0_v.at[pl.ds(0, RPT % C)],
                    acc_sh.at[pl.ds(rbase + (RPT // C) * C, RPT % C)])
    plsc.subcore_barrier()

    def start_gather(i, rows, sem):
        pltpu.async_copy(hs_hbm.at[sidx_v.at[i]], rows, sem)

    def wait_gather(i, rows, sem):
        pltpu.make_async_copy(hs_hbm.at[sidx_v.at[i]], rows, sem).wait()

    def start_didx(i, didx, sem):
        pltpu.async_copy(dst_hbm.at[pl.ds(ebase + i * C, C)], didx, sem)

    def wait_didx(i, didx, sem):
        pltpu.make_async_copy(dst_hbm.at[pl.ds(ebase + i * C, C)], didx,
                              sem).wait()

    def start_scat(rows, didx, sem):
        pltpu.async_copy(rows, acc_sh.at[didx], sem, add=True)

    def wait_scat(rows, didx, sem):
        pltpu.make_async_copy(rows, acc_sh.at[didx], sem).wait()

    start_didx(0, didx0_v, d0)
    start_didx(1, didx1_v, d1)
    start_gather(0, rows0_v, g0)
    start_gather(1, rows1_v, g1)

    def chunk(j, carry):
        i0 = j * 2
        wait_gather(i0, rows0_v, g0)
        wait_didx(i0, didx0_v, d0)
        start_scat(rows0_v, didx0_v, s0)
        wait_gather(i0 + 1, rows1_v, g1)
        wait_didx(i0 + 1, didx1_v, d1)
        start_scat(rows1_v, didx1_v, s1)
        wait_scat(rows0_v, didx0_v, s0)
        start_gather(i0 + 2, rows0_v, g0)
        start_didx(i0 + 2, didx0_v, d0)
        wait_scat(rows1_v, didx1_v, s1)
        start_gather(i0 + 3, rows1_v, g1)
        start_didx(i0 + 3, didx1_v, d1)
        return carry
    last = CHUNKS_PER_TILE - 2
    lax.fori_loop(0, CHUNKS_PER_TILE // 2 - 1, chunk, 0)
    wait_gather(last, rows0_v, g0)
    wait_didx(last, didx0_v, d0)
    start_scat(rows0_v, didx0_v, s0)
    wait_gather(last + 1, rows1_v, g1)
    wait_didx(last + 1, didx1_v, d1)
    start_scat(rows1_v, didx1_v, s1)
    wait_scat(rows0_v, didx0_v, s0)
    wait_scat(rows1_v, didx1_v, s1)
    plsc.subcore_barrier()

    def ocopy(i, carry):
        ob = pl.multiple_of(rbase + i * C, 8)
        pltpu.sync_copy(acc_sh.at[pl.ds(ob, C)], rows0_v)
        pltpu.sync_copy(rows0_v, out_hbm.at[c, pl.ds(ob, C)])
        return carry
    lax.fori_loop(0, RPT // C, ocopy, 0)
    ot = pl.multiple_of(rbase + (RPT // C) * C, 8)
    pltpu.sync_copy(acc_sh.at[pl.ds(ot, RPT % C)], rows0_v.at[pl.ds(0, RPT % C)])
    pltpu.sync_copy(rows0_v.at[pl.ds(0, RPT % C)], out_hbm.at[c, pl.ds(ot, RPT % C)])


def _make_agg(d, tc_tiling=True):
    return pl.kernel(
        functools.partial(_agg_body, d=d),
        out_type=jax.ShapeDtypeStruct((NC, NACC, d), jnp.float32),
        mesh=_MESH,
        compiler_params=pltpu.CompilerParams(use_tc_tiling_on_sc=tc_tiling),
        scratch_types=[
            pltpu.VMEM((CHUNKS_PER_TILE, C), jnp.int32),
            pltpu.VMEM((C,), jnp.int32),
            pltpu.VMEM((C,), jnp.int32),
            pltpu.VMEM((C, d), jnp.float32),
            pltpu.VMEM((C, d), jnp.float32),
            pltpu.VMEM_SHARED((NACC, d), jnp.float32),
            pltpu.SemaphoreType.DMA,
            pltpu.SemaphoreType.DMA,
            pltpu.SemaphoreType.DMA,
            pltpu.SemaphoreType.DMA,
            pltpu.SemaphoreType.DMA,
            pltpu.SemaphoreType.DMA,
        ],
    )


def _agg4_body(hs_hbm, src_hbm, dst_hbm, out_hbm,
               sidx_v, di0, di1, di2, di3, ro0, ro1, ro2, ro3, acc_sh,
               g0, g1, g2, g3, s0, s1, s2, s3, f0, f1, f2, f3, d):
    c = lax.axis_index("c")
    s = lax.axis_index("s")
    wid = c * NS + s
    rbase = pl.multiple_of(s * RPT, 8)
    cbase = pl.multiple_of(wid * CHUNKS_PER_TILE, 8)
    ebase = pl.multiple_of(wid * EW, 8)
    didx = [di0, di1, di2, di3]
    rows = [ro0, ro1, ro2, ro3]
    gs = [g0, g1, g2, g3]
    ss = [s0, s1, s2, s3]
    fs = [f0, f1, f2, f3]

    pltpu.sync_copy(src_hbm.at[pl.ds(cbase, CHUNKS_PER_TILE)], sidx_v)

    def zrow(i, carry):
        for j in range(d // 16):
            ro0[i, pl.ds(j * 16, 16)] = jnp.zeros((16,), jnp.float32)
        return carry
    lax.fori_loop(0, C, zrow, 0)

    def zcopy(i, carry):
        pltpu.sync_copy(ro0, acc_sh.at[pl.ds(rbase + i * C, C)])
        return carry
    lax.fori_loop(0, RPT // C, zcopy, 0)
    pltpu.sync_copy(ro0.at[pl.ds(0, RPT % C)],
                    acc_sh.at[pl.ds(rbase + (RPT // C) * C, RPT % C)])
    plsc.subcore_barrier()

    def start_gather(i, b):
        pltpu.async_copy(hs_hbm.at[sidx_v.at[i]], rows[b], gs[b])

    def wait_gather(i, b):
        pltpu.make_async_copy(hs_hbm.at[sidx_v.at[i]], rows[b], gs[b]).wait()

    def start_didx(i, b):
        pltpu.async_copy(dst_hbm.at[pl.ds(ebase + i * C, C)], didx[b], fs[b])

    def wait_didx(i, b):
        pltpu.make_async_copy(dst_hbm.at[pl.ds(ebase + i * C, C)], didx[b],
                              fs[b]).wait()

    def start_scat(b):
        pltpu.async_copy(rows[b], acc_sh.at[didx[b]], ss[b], add=True)

    def wait_scat(b):
        pltpu.make_async_copy(rows[b], acc_sh.at[didx[b]], ss[b]).wait()

    for b in range(4):
        start_didx(b, b)
        start_gather(b, b)

    def quad(j, carry):
        i0 = j * 4
        for b in range(4):
            wait_gather(i0 + b, b)
            wait_didx(i0 + b, b)
            start_scat(b)
        for b in range(4):
            wait_scat(b)
            start_gather(i0 + 4 + b, b)
            start_didx(i0 + 4 + b, b)
        return carry
    lax.fori_loop(0, CHUNKS_PER_TILE // 4 - 1, quad, 0)
    last = CHUNKS_PER_TILE - 4
    for b in range(4):
        wait_gather(last + b, b)
        wait_didx(last + b, b)
        start_scat(b)
    for b in range(4):
        wait_scat(b)
    plsc.subcore_barrier()

    def ocopy(i, carry):
        ob = pl.multiple_of(rbase + i * C, 8)
        pltpu.sync_copy(acc_sh.at[pl.ds(ob, C)], ro1)
        pltpu.sync_copy(ro1, out_hbm.at[c, pl.ds(ob, C)])
        return carry
    lax.fori_loop(0, RPT // C, ocopy, 0)
    ot = pl.multiple_of(rbase + (RPT // C) * C, 8)
    pltpu.sync_copy(acc_sh.at[pl.ds(ot, RPT % C)], ro1.at[pl.ds(0, RPT % C)])
    pltpu.sync_copy(ro1.at[pl.ds(0, RPT % C)], out_hbm.at[c, pl.ds(ot, RPT % C)])


def _make_agg4(d, tc_tiling=True):
    return pl.kernel(
        functools.partial(_agg4_body, d=d),
        out_type=jax.ShapeDtypeStruct((NC, NACC, d), jnp.float32),
        mesh=_MESH,
        compiler_params=pltpu.CompilerParams(use_tc_tiling_on_sc=tc_tiling),
        scratch_types=(
            [pltpu.VMEM((CHUNKS_PER_TILE, C), jnp.int32)]
            + [pltpu.VMEM((C,), jnp.int32) for _ in range(4)]
            + [pltpu.VMEM((C, d), jnp.float32) for _ in range(4)]
            + [pltpu.VMEM_SHARED((NACC, d), jnp.float32)]
            + [pltpu.SemaphoreType.DMA for _ in range(12)]
        ),
    )


_agg128 = _make_agg(HID)
_agg64 = _make_agg4(OUT, tc_tiling=False)

BM = 400
NROW = NACC // 128


def _m1_body(x_ref, w_ref, dv_ref, hs_ref):
    h = jnp.dot(x_ref[...], w_ref[...], preferred_element_type=jnp.float32)
    hs_ref[...] = h * dv_ref[...]


_m1_call = pl.pallas_call(
    _m1_body,
    grid=(N // BM,),
    in_specs=[
        pl.BlockSpec((BM, F_IN), lambda i: (i, 0)),
        pl.BlockSpec((F_IN, HID), lambda i: (0, 0)),
        pl.BlockSpec((BM, HID), lambda i: (i, 0)),
    ],
    out_specs=pl.BlockSpec((BM, HID), lambda i: (i, 0)),
    out_shape=jax.ShapeDtypeStruct((N, HID), jnp.float32),
)


def _l2_body(a0_ref, a1_ref, hs1_ref, dv_ref, b1_ref, w2_ref, o_ref):
    act = (a0_ref[0] + a1_ref[0] + hs1_ref[...]) * dv_ref[...] + b1_ref[...]
    act = jnp.maximum(act, 0.0)
    h2 = jnp.dot(act, w2_ref[...], preferred_element_type=jnp.float32)
    o_ref[...] = h2 * dv_ref[:, :OUT]


_l2_call = pl.pallas_call(
    _l2_body,
    grid=(N // BM,),
    in_specs=[
        pl.BlockSpec((1, BM, HID), lambda i: (0, i, 0)),
        pl.BlockSpec((1, BM, HID), lambda i: (1, i, 0)),
        pl.BlockSpec((BM, HID), lambda i: (i, 0)),
        pl.BlockSpec((BM, HID), lambda i: (i, 0)),
        pl.BlockSpec((1, HID), lambda i: (0, 0)),
        pl.BlockSpec((HID, OUT), lambda i: (0, 0)),
    ],
    out_specs=pl.BlockSpec((BM, OUT), lambda i: (i, 0)),
    out_shape=jax.ShapeDtypeStruct((N, OUT), jnp.float32),
)


def _fin_body(a0_ref, a1_ref, hs2_ref, dv_ref, b2_ref, o_ref):
    t = a0_ref[0] + a1_ref[0] + hs2_ref[...]
    o = t * dv_ref[:, :OUT] + b2_ref[...]
    o_ref[...] = jax.nn.sigmoid(o)


_fin_call = pl.pallas_call(
    _fin_body,
    grid=(N // BM,),
    in_specs=[
        pl.BlockSpec((1, BM, OUT), lambda i: (0, i, 0)),
        pl.BlockSpec((1, BM, OUT), lambda i: (1, i, 0)),
        pl.BlockSpec((BM, OUT), lambda i: (i, 0)),
        pl.BlockSpec((BM, HID), lambda i: (i, 0)),
        pl.BlockSpec((1, OUT), lambda i: (0, 0)),
    ],
    out_specs=pl.BlockSpec((BM, OUT), lambda i: (i, 0)),
    out_shape=jax.ShapeDtypeStruct((N, OUT), jnp.float32),
)


def kernel(x, edge_index, W1, b1, W2, b2):
    src = edge_index[0]
    dst = edge_index[1]
    pad_i = jnp.arange(PAD, dtype=jnp.int32)
    src_p = jnp.concatenate([src, (pad_i * 131) % N]).reshape(-1, C)
    dst_f = jnp.concatenate([dst, N + (pad_i % TRASH)])
    dst_p = dst_f.reshape(-1, C)

    degf = _deg_call(dst_p)
    deg = degf[:N] + degf[NACC:NACC + N] + 1.0
    dinv = jnp.broadcast_to(lax.rsqrt(deg)[:, None], (N, HID))

    hs1 = _m1_call(x, W1, dinv)
    agg1 = _agg128(hs1, src_p, dst_f)
    hs2 = _l2_call(agg1, agg1, hs1, dinv, b1[None, :], W2)
    agg2 = _agg64(hs2, src_p, dst_f)
    return _fin_call(agg2, agg2, hs2, dinv, b2[None, :])


# speedup vs baseline: 1.2230x; 1.1006x over previous
import functools

import jax
import jax.numpy as jnp
from jax import lax
from jax.experimental import pallas as pl
from jax.experimental.pallas import tpu as pltpu
from jax.experimental.pallas import tpu_sc as plsc

N = 10000
E = 320000
F_IN = 128
HID = 128
OUT = 64

NC = 2
NS = 16
NW = NC * NS

C = 128
CHUNKS_PER_TILE = 80
EW = CHUNKS_PER_TILE * C
E_PAD = NW * EW
PAD = E_PAD - E

TRASH = 512
NACC = 10752
RPT = NACC // NS
ZB = 96

_MESH = plsc.VectorSubcoreMesh(
    core_axis_name="c", subcore_axis_name="s", num_cores=NC, num_subcores=NS)


def _zero_vmem_1d(ref, n):
    def body(i, carry):
        ref[pl.ds(i * 16, 16)] = jnp.zeros((16,), jnp.float32)
        return carry
    lax.fori_loop(0, n // 16, body, 0)


def _deg_body(dst_hbm, out_hbm, didx_v, ones_v, zeros_v, acc_sh, sem):
    del sem
    c = lax.axis_index("c")
    s = lax.axis_index("s")
    wid = c * NS + s
    rbase = pl.multiple_of(s * RPT, 8)

    pltpu.sync_copy(
        dst_hbm.at[pl.ds(pl.multiple_of(wid * CHUNKS_PER_TILE, 8),
                         CHUNKS_PER_TILE)], didx_v)

    def fill_ones(i, carry):
        ones_v[pl.ds(i * 16, 16)] = jnp.full((16,), 1.0, jnp.float32)
        return carry
    lax.fori_loop(0, C // 16, fill_ones, 0)
    _zero_vmem_1d(zeros_v, RPT)
    pltpu.sync_copy(zeros_v, acc_sh.at[pl.ds(rbase, RPT)])
    plsc.subcore_barrier()

    def chunk(i, carry):
        pltpu.sync_copy(ones_v, acc_sh.at[didx_v.at[i]], add=True)
        return carry
    lax.fori_loop(0, CHUNKS_PER_TILE, chunk, 0)
    plsc.subcore_barrier()
    obase = pl.multiple_of(c * NACC + s * RPT, 8)
    pltpu.sync_copy(acc_sh.at[pl.ds(rbase, RPT)], zeros_v)
    pltpu.sync_copy(zeros_v, out_hbm.at[pl.ds(obase, RPT)])


_deg_call = pl.kernel(
    _deg_body,
    out_type=jax.ShapeDtypeStruct((NC * NACC,), jnp.float32),
    mesh=_MESH,
    scratch_types=[
        pltpu.VMEM((CHUNKS_PER_TILE, C), jnp.int32),
        pltpu.VMEM((C,), jnp.float32),
        pltpu.VMEM((RPT,), jnp.float32),
        pltpu.VMEM_SHARED((NACC,), jnp.float32),
        pltpu.SemaphoreType.DMA,
    ],
)


NB = 4


def _aggn_body(hs_hbm, src_hbm, dst_hbm, out_hbm, *scr, d, cc):
    sidx_v = scr[0]
    didx = list(scr[1:1 + NB])
    rows = list(scr[1 + NB:1 + 2 * NB])
    acc_sh = scr[1 + 2 * NB]
    gs = list(scr[2 + 2 * NB:2 + 3 * NB])
    ss = list(scr[2 + 3 * NB:2 + 4 * NB])
    fs = list(scr[2 + 4 * NB:2 + 5 * NB])
    chunks = EW // cc

    c = lax.axis_index("c")
    s = lax.axis_index("s")
    wid = c * NS + s
    rbase = pl.multiple_of(s * RPT, 8)
    ebase = pl.multiple_of(wid * EW, 8)

    pltpu.sync_copy(src_hbm.at[pl.ds(ebase, EW)], sidx_v)

    def zrow(i, carry):
        for j in range(d // 16):
            rows[0][i, pl.ds(j * 16, 16)] = jnp.zeros((16,), jnp.float32)
        return carry
    lax.fori_loop(0, cc, zrow, 0)

    def zcopy(i, carry):
        pltpu.sync_copy(rows[0], acc_sh.at[pl.ds(rbase + i * cc, cc)])
        return carry
    lax.fori_loop(0, RPT // cc, zcopy, 0)
    if RPT % cc:
        pltpu.sync_copy(rows[0].at[pl.ds(0, RPT % cc)],
                        acc_sh.at[pl.ds(rbase + (RPT // cc) * cc, RPT % cc)])
    plsc.subcore_barrier()

    def start_gather(i, b):
        pltpu.async_copy(hs_hbm.at[sidx_v.at[pl.ds(i * cc, cc)]], rows[b],
                         gs[b])

    def wait_gather(i, b):
        pltpu.make_async_copy(hs_hbm.at[sidx_v.at[pl.ds(i * cc, cc)]],
                              rows[b], gs[b]).wait()

    def start_didx(i, b):
        pltpu.async_copy(dst_hbm.at[pl.ds(ebase + i * cc, cc)], didx[b],
                         fs[b])

    def wait_didx(i, b):
        pltpu.make_async_copy(dst_hbm.at[pl.ds(ebase + i * cc, cc)], didx[b],
                              fs[b]).wait()

    def start_scat(b):
        pltpu.async_copy(rows[b], acc_sh.at[didx[b]], ss[b], add=True)

    def wait_scat(b):
        pltpu.make_async_copy(rows[b], acc_sh.at[didx[b]], ss[b]).wait()

    for b in range(NB):
        start_didx(b, b)
        start_gather(b, b)

    def ring(j, carry):
        i0 = j * NB
        for b in range(NB):
            wait_gather(i0 + b, b)
            wait_didx(i0 + b, b)
            start_scat(b)
        for b in range(NB):
            wait_scat(b)
            start_gather(i0 + NB + b, b)
            start_didx(i0 + NB + b, b)
        return carry
    lax.fori_loop(0, chunks // NB - 1, ring, 0)
    last = chunks - NB
    for b in range(NB):
        wait_gather(last + b, b)
        wait_didx(last + b, b)
        start_scat(b)
    for b in range(NB):
        wait_scat(b)
    plsc.subcore_barrier()

    def ocopy(i, carry):
        ob = pl.multiple_of(rbase + i * cc, 8)
        pltpu.sync_copy(acc_sh.at[pl.ds(ob, cc)], rows[1])
        pltpu.sync_copy(rows[1], out_hbm.at[c, pl.ds(ob, cc)])
        return carry
    lax.fori_loop(0, RPT // cc, ocopy, 0)
    if RPT % cc:
        ot = pl.multiple_of(rbase + (RPT // cc) * cc, 8)
        pltpu.sync_copy(acc_sh.at[pl.ds(ot, RPT % cc)],
                        rows[1].at[pl.ds(0, RPT % cc)])
        pltpu.sync_copy(rows[1].at[pl.ds(0, RPT % cc)],
                        out_hbm.at[c, pl.ds(ot, RPT % cc)])


def _make_agg(d, cc, tc_tiling=True):
    return pl.kernel(
        functools.partial(_aggn_body, d=d, cc=cc),
        out_type=jax.ShapeDtypeStruct((NC, NACC, d), jnp.float32),
        mesh=_MESH,
        compiler_params=pltpu.CompilerParams(use_tc_tiling_on_sc=tc_tiling),
        scratch_types=(
            [pltpu.VMEM((EW,), jnp.int32)]
            + [pltpu.VMEM((cc,), jnp.int32) for _ in range(NB)]
            + [pltpu.VMEM((cc, d), jnp.float32) for _ in range(NB)]
            + [pltpu.VMEM_SHARED((NACC, d), jnp.float32)]
            + [pltpu.SemaphoreType.DMA for _ in range(3 * NB)]
        ),
    )


_agg128 = _make_agg(HID, 64)
_agg64 = _make_agg(OUT, C, tc_tiling=False)

BM = 400
NROW = NACC // 128


def _m1_body(x_ref, w_ref, dv_ref, hs_ref):
    h = jnp.dot(x_ref[...], w_ref[...], preferred_element_type=jnp.float32)
    hs_ref[...] = h * dv_ref[...]


_m1_call = pl.pallas_call(
    _m1_body,
    grid=(N // BM,),
    in_specs=[
        pl.BlockSpec((BM, F_IN), lambda i: (i, 0)),
        pl.BlockSpec((F_IN, HID), lambda i: (0, 0)),
        pl.BlockSpec((BM, HID), lambda i: (i, 0)),
    ],
    out_specs=pl.BlockSpec((BM, HID), lambda i: (i, 0)),
    out_shape=jax.ShapeDtypeStruct((N, HID), jnp.float32),
)


def _l2_body(a0_ref, a1_ref, hs1_ref, dv_ref, b1_ref, w2_ref, o_ref):
    act = (a0_ref[0] + a1_ref[0] + hs1_ref[...]) * dv_ref[...] + b1_ref[...]
    act = jnp.maximum(act, 0.0)
    h2 = jnp.dot(act, w2_ref[...], preferred_element_type=jnp.float32)
    o_ref[...] = h2 * dv_ref[:, :OUT]


_l2_call = pl.pallas_call(
    _l2_body,
    grid=(N // BM,),
    in_specs=[
        pl.BlockSpec((1, BM, HID), lambda i: (0, i, 0)),
        pl.BlockSpec((1, BM, HID), lambda i: (1, i, 0)),
        pl.BlockSpec((BM, HID), lambda i: (i, 0)),
        pl.BlockSpec((BM, HID), lambda i: (i, 0)),
        pl.BlockSpec((1, HID), lambda i: (0, 0)),
        pl.BlockSpec((HID, OUT), lambda i: (0, 0)),
    ],
    out_specs=pl.BlockSpec((BM, OUT), lambda i: (i, 0)),
    out_shape=jax.ShapeDtypeStruct((N, OUT), jnp.float32),
)


def _fin_body(a0_ref, a1_ref, hs2_ref, dv_ref, b2_ref, o_ref):
    t = a0_ref[0] + a1_ref[0] + hs2_ref[...]
    o = t * dv_ref[:, :OUT] + b2_ref[...]
    o_ref[...] = jax.nn.sigmoid(o)


_fin_call = pl.pallas_call(
    _fin_body,
    grid=(N // BM,),
    in_specs=[
        pl.BlockSpec((1, BM, OUT), lambda i: (0, i, 0)),
        pl.BlockSpec((1, BM, OUT), lambda i: (1, i, 0)),
        pl.BlockSpec((BM, OUT), lambda i: (i, 0)),
        pl.BlockSpec((BM, HID), lambda i: (i, 0)),
        pl.BlockSpec((1, OUT), lambda i: (0, 0)),
    ],
    out_specs=pl.BlockSpec((BM, OUT), lambda i: (i, 0)),
    out_shape=jax.ShapeDtypeStruct((N, OUT), jnp.float32),
)


def kernel(x, edge_index, W1, b1, W2, b2):
    src = edge_index[0]
    dst = edge_index[1]
    pad_i = jnp.arange(PAD, dtype=jnp.int32)
    src_p = jnp.concatenate([src, (pad_i * 131) % N])
    dst_f = jnp.concatenate([dst, N + (pad_i % TRASH)])
    dst_p = dst_f.reshape(-1, C)

    degf = _deg_call(dst_p)
    deg = degf[:N] + degf[NACC:NACC + N] + 1.0
    dinv = jnp.broadcast_to(lax.rsqrt(deg)[:, None], (N, HID))

    hs1 = _m1_call(x, W1, dinv)
    agg1 = _agg128(hs1, src_p, dst_f)
    hs2 = _l2_call(agg1, agg1, hs1, dinv, b1[None, :], W2)
    agg2 = _agg64(hs2, src_p, dst_f)
    return _fin_call(agg2, agg2, hs2, dinv, b2[None, :])


# speedup vs baseline: 1.2257x; 1.0023x over previous
import functools

import jax
import jax.numpy as jnp
from jax import lax
from jax.experimental import pallas as pl
from jax.experimental.pallas import tpu as pltpu
from jax.experimental.pallas import tpu_sc as plsc

N = 10000
E = 320000
F_IN = 128
HID = 128
OUT = 64

NC = 2
NS = 16
NW = NC * NS

C = 128
CHUNKS_PER_TILE = 80
EW = CHUNKS_PER_TILE * C
E_PAD = NW * EW
PAD = E_PAD - E

TRASH = 512
NACC = 10752
RPT = NACC // NS
ZB = 96

_MESH = plsc.VectorSubcoreMesh(
    core_axis_name="c", subcore_axis_name="s", num_cores=NC, num_subcores=NS)


def _zero_vmem_1d(ref, n):
    def body(i, carry):
        ref[pl.ds(i * 16, 16)] = jnp.zeros((16,), jnp.float32)
        return carry
    lax.fori_loop(0, n // 16, body, 0)


def _deg_body(dst_hbm, out_hbm, didx_v, ones_v, zeros_v, acc_sh, sem):
    del sem
    c = lax.axis_index("c")
    s = lax.axis_index("s")
    wid = c * NS + s
    rbase = pl.multiple_of(s * RPT, 8)

    pltpu.sync_copy(
        dst_hbm.at[pl.ds(pl.multiple_of(wid * CHUNKS_PER_TILE, 8),
                         CHUNKS_PER_TILE)], didx_v)

    def fill_ones(i, carry):
        ones_v[pl.ds(i * 16, 16)] = jnp.full((16,), 1.0, jnp.float32)
        return carry
    lax.fori_loop(0, C // 16, fill_ones, 0)
    _zero_vmem_1d(zeros_v, RPT)
    pltpu.sync_copy(zeros_v, acc_sh.at[pl.ds(rbase, RPT)])
    plsc.subcore_barrier()

    def chunk(i, carry):
        pltpu.sync_copy(ones_v, acc_sh.at[didx_v.at[i]], add=True)
        return carry
    lax.fori_loop(0, CHUNKS_PER_TILE, chunk, 0)
    plsc.subcore_barrier()
    obase = pl.multiple_of(c * NACC + s * RPT, 8)
    pltpu.sync_copy(acc_sh.at[pl.ds(rbase, RPT)], zeros_v)
    pltpu.sync_copy(zeros_v, out_hbm.at[pl.ds(obase, RPT)])


_deg_call = pl.kernel(
    _deg_body,
    out_type=jax.ShapeDtypeStruct((NC * NACC,), jnp.float32),
    mesh=_MESH,
    scratch_types=[
        pltpu.VMEM((CHUNKS_PER_TILE, C), jnp.int32),
        pltpu.VMEM((C,), jnp.float32),
        pltpu.VMEM((RPT,), jnp.float32),
        pltpu.VMEM_SHARED((NACC,), jnp.float32),
        pltpu.SemaphoreType.DMA,
    ],
)


def _aggn_body(hs_hbm, src_hbm, dst_hbm, out_hbm, *scr, d, cc, NB):
    sidx_v = scr[0]
    didx = list(scr[1:1 + NB])
    rows = list(scr[1 + NB:1 + 2 * NB])
    acc_sh = scr[1 + 2 * NB]
    gs = list(scr[2 + 2 * NB:2 + 3 * NB])
    ss = list(scr[2 + 3 * NB:2 + 4 * NB])
    fs = list(scr[2 + 4 * NB:2 + 5 * NB])
    chunks = EW // cc

    c = lax.axis_index("c")
    s = lax.axis_index("s")
    wid = c * NS + s
    rbase = pl.multiple_of(s * RPT, 8)
    ebase = pl.multiple_of(wid * EW, 8)

    pltpu.sync_copy(src_hbm.at[pl.ds(ebase, EW)], sidx_v)

    def zrow(i, carry):
        for j in range(d // 16):
            rows[0][i, pl.ds(j * 16, 16)] = jnp.zeros((16,), jnp.float32)
        return carry
    lax.fori_loop(0, cc, zrow, 0)

    def zcopy(i, carry):
        pltpu.sync_copy(rows[0], acc_sh.at[pl.ds(rbase + i * cc, cc)])
        return carry
    lax.fori_loop(0, RPT // cc, zcopy, 0)
    if RPT % cc:
        pltpu.sync_copy(rows[0].at[pl.ds(0, RPT % cc)],
                        acc_sh.at[pl.ds(rbase + (RPT // cc) * cc, RPT % cc)])
    plsc.subcore_barrier()

    def start_gather(i, b):
        pltpu.async_copy(hs_hbm.at[sidx_v.at[pl.ds(i * cc, cc)]], rows[b],
                         gs[b])

    def wait_gather(i, b):
        pltpu.make_async_copy(hs_hbm.at[sidx_v.at[pl.ds(i * cc, cc)]],
                              rows[b], gs[b]).wait()

    def start_didx(i, b):
        pltpu.async_copy(dst_hbm.at[pl.ds(ebase + i * cc, cc)], didx[b],
                         fs[b])

    def wait_didx(i, b):
        pltpu.make_async_copy(dst_hbm.at[pl.ds(ebase + i * cc, cc)], didx[b],
                              fs[b]).wait()

    def start_scat(b):
        pltpu.async_copy(rows[b], acc_sh.at[didx[b]], ss[b], add=True)

    def wait_scat(b):
        pltpu.make_async_copy(rows[b], acc_sh.at[didx[b]], ss[b]).wait()

    for b in range(NB):
        start_didx(b, b)
        start_gather(b, b)

    def ring(j, carry):
        i0 = j * NB
        for b in range(NB):
            wait_gather(i0 + b, b)
            wait_didx(i0 + b, b)
            start_scat(b)
        for b in range(NB):
            wait_scat(b)
            start_gather(i0 + NB + b, b)
            start_didx(i0 + NB + b, b)
        return carry
    lax.fori_loop(0, chunks // NB - 1, ring, 0)
    last = chunks - NB
    for b in range(NB):
        wait_gather(last + b, b)
        wait_didx(last + b, b)
        start_scat(b)
    for b in range(NB):
        wait_scat(b)
    plsc.subcore_barrier()

    def ocopy(i, carry):
        ob = pl.multiple_of(rbase + i * cc, 8)
        pltpu.sync_copy(acc_sh.at[pl.ds(ob, cc)], rows[1])
        pltpu.sync_copy(rows[1], out_hbm.at[c, pl.ds(ob, cc)])
        return carry
    lax.fori_loop(0, RPT // cc, ocopy, 0)
    if RPT % cc:
        ot = pl.multiple_of(rbase + (RPT // cc) * cc, 8)
        pltpu.sync_copy(acc_sh.at[pl.ds(ot, RPT % cc)],
                        rows[1].at[pl.ds(0, RPT % cc)])
        pltpu.sync_copy(rows[1].at[pl.ds(0, RPT % cc)],
                        out_hbm.at[c, pl.ds(ot, RPT % cc)])


def _make_agg(d, cc, NB, tc_tiling=True):
    return pl.kernel(
        functools.partial(_aggn_body, d=d, cc=cc, NB=NB),
        out_type=jax.ShapeDtypeStruct((NC, NACC, d), jnp.float32),
        mesh=_MESH,
        compiler_params=pltpu.CompilerParams(use_tc_tiling_on_sc=tc_tiling),
        scratch_types=(
            [pltpu.VMEM((EW,), jnp.int32)]
            + [pltpu.VMEM((cc,), jnp.int32) for _ in range(NB)]
            + [pltpu.VMEM((cc, d), jnp.float32) for _ in range(NB)]
            + [pltpu.VMEM_SHARED((NACC, d), jnp.float32)]
            + [pltpu.SemaphoreType.DMA for _ in range(3 * NB)]
        ),
    )


_agg128 = _make_agg(HID, 32, 8)
_agg64 = _make_agg(OUT, C, 8, tc_tiling=False)

BM = 400
NROW = NACC // 128


def _m1_body(x_ref, w_ref, dv_ref, hs_ref):
    h = jnp.dot(x_ref[...], w_ref[...], preferred_element_type=jnp.float32)
    hs_ref[...] = h * dv_ref[...]


_m1_call = pl.pallas_call(
    _m1_body,
    grid=(N // BM,),
    in_specs=[
        pl.BlockSpec((BM, F_IN), lambda i: (i, 0)),
        pl.BlockSpec((F_IN, HID), lambda i: (0, 0)),
        pl.BlockSpec((BM, HID), lambda i: (i, 0)),
    ],
    out_specs=pl.BlockSpec((BM, HID), lambda i: (i, 0)),
    out_shape=jax.ShapeDtypeStruct((N, HID), jnp.float32),
)


def _l2_body(a0_ref, a1_ref, hs1_ref, dv_ref, b1_ref, w2_ref, o_ref):
    act = (a0_ref[0] + a1_ref[0] + hs1_ref[...]) * dv_ref[...] + b1_ref[...]
    act = jnp.maximum(act, 0.0)
    h2 = jnp.dot(act, w2_ref[...], preferred_element_type=jnp.float32)
    o_ref[...] = h2 * dv_ref[:, :OUT]


_l2_call = pl.pallas_call(
    _l2_body,
    grid=(N // BM,),
    in_specs=[
        pl.BlockSpec((1, BM, HID), lambda i: (0, i, 0)),
        pl.BlockSpec((1, BM, HID), lambda i: (1, i, 0)),
        pl.BlockSpec((BM, HID), lambda i: (i, 0)),
        pl.BlockSpec((BM, HID), lambda i: (i, 0)),
        pl.BlockSpec((1, HID), lambda i: (0, 0)),
        pl.BlockSpec((HID, OUT), lambda i: (0, 0)),
    ],
    out_specs=pl.BlockSpec((BM, OUT), lambda i: (i, 0)),
    out_shape=jax.ShapeDtypeStruct((N, OUT), jnp.float32),
)


def _fin_body(a0_ref, a1_ref, hs2_ref, dv_ref, b2_ref, o_ref):
    t = a0_ref[0] + a1_ref[0] + hs2_ref[...]
    o = t * dv_ref[:, :OUT] + b2_ref[...]
    o_ref[...] = jax.nn.sigmoid(o)


_fin_call = pl.pallas_call(
    _fin_body,
    grid=(N // BM,),
    in_specs=[
        pl.BlockSpec((1, BM, OUT), lambda i: (0, i, 0)),
        pl.BlockSpec((1, BM, OUT), lambda i: (1, i, 0)),
        pl.BlockSpec((BM, OUT), lambda i: (i, 0)),
        pl.BlockSpec((BM, HID), lambda i: (i, 0)),
        pl.BlockSpec((1, OUT), lambda i: (0, 0)),
    ],
    out_specs=pl.BlockSpec((BM, OUT), lambda i: (i, 0)),
    out_shape=jax.ShapeDtypeStruct((N, OUT), jnp.float32),
)


def kernel(x, edge_index, W1, b1, W2, b2):
    src = edge_index[0]
    dst = edge_index[1]
    pad_i = jnp.arange(PAD, dtype=jnp.int32)
    src_p = jnp.concatenate([src, (pad_i * 131) % N])
    dst_f = jnp.concatenate([dst, N + (pad_i % TRASH)])
    dst_p = dst_f.reshape(-1, C)

    degf = _deg_call(dst_p)
    deg = degf[:N] + degf[NACC:NACC + N] + 1.0
    dinv = jnp.broadcast_to(lax.rsqrt(deg)[:, None], (N, HID))

    hs1 = _m1_call(x, W1, dinv)
    agg1 = _agg128(hs1, src_p, dst_f)
    hs2 = _l2_call(agg1, agg1, hs1, dinv, b1[None, :], W2)
    agg2 = _agg64(hs2, src_p, dst_f)
    return _fin_call(agg2, agg2, hs2, dinv, b2[None, :])


# speedup vs baseline: 1.2280x; 1.0019x over previous
import functools

import jax
import jax.numpy as jnp
from jax import lax
from jax.experimental import pallas as pl
from jax.experimental.pallas import tpu as pltpu
from jax.experimental.pallas import tpu_sc as plsc

N = 10000
E = 320000
F_IN = 128
HID = 128
OUT = 64

NC = 2
NS = 16
NW = NC * NS

C = 128
CHUNKS_PER_TILE = 80
EW = CHUNKS_PER_TILE * C
E_PAD = NW * EW
PAD = E_PAD - E

TRASH = 512
NACC = 10752
RPT = NACC // NS

_MESH = plsc.VectorSubcoreMesh(
    core_axis_name="c", subcore_axis_name="s", num_cores=NC, num_subcores=NS)


def _zero_vmem_1d(ref, n):
    def body(i, carry):
        ref[pl.ds(i * 16, 16)] = jnp.zeros((16,), jnp.float32)
        return carry
    lax.fori_loop(0, n // 16, body, 0)


def _deg_body(dst_hbm, out_hbm, didx_v, ones_v, zeros_v, acc_sh, sem):
    del sem
    c = lax.axis_index("c")
    s = lax.axis_index("s")
    wid = c * NS + s
    rbase = pl.multiple_of(s * RPT, 8)

    pltpu.sync_copy(
        dst_hbm.at[pl.ds(pl.multiple_of(wid * CHUNKS_PER_TILE, 8),
                         CHUNKS_PER_TILE)], didx_v)

    def fill_ones(i, carry):
        ones_v[pl.ds(i * 16, 16)] = jnp.full((16,), 1.0, jnp.float32)
        return carry
    lax.fori_loop(0, C // 16, fill_ones, 0)
    _zero_vmem_1d(zeros_v, RPT)
    pltpu.sync_copy(zeros_v, acc_sh.at[pl.ds(rbase, RPT)])
    plsc.subcore_barrier()

    def chunk(i, carry):
        pltpu.sync_copy(ones_v, acc_sh.at[didx_v.at[i]], add=True)
        return carry
    lax.fori_loop(0, CHUNKS_PER_TILE, chunk, 0)
    plsc.subcore_barrier()
    obase = pl.multiple_of(c * NACC + s * RPT, 8)
    pltpu.sync_copy(acc_sh.at[pl.ds(rbase, RPT)], zeros_v)
    pltpu.sync_copy(zeros_v, out_hbm.at[pl.ds(obase, RPT)])


_deg_call = pl.kernel(
    _deg_body,
    out_type=jax.ShapeDtypeStruct((NC * NACC,), jnp.float32),
    mesh=_MESH,
    scratch_types=[
        pltpu.VMEM((CHUNKS_PER_TILE, C), jnp.int32),
        pltpu.VMEM((C,), jnp.float32),
        pltpu.VMEM((RPT,), jnp.float32),
        pltpu.VMEM_SHARED((NACC,), jnp.float32),
        pltpu.SemaphoreType.DMA,
    ],
)


def _aggn_body(hs_hbm, src_hbm, dst_hbm, out_hbm, *scr, d, cc, NB):
    sidx_v = scr[0]
    didx = list(scr[1:1 + NB])
    rows = list(scr[1 + NB:1 + 2 * NB])
    acc_sh = scr[1 + 2 * NB]
    gs = list(scr[2 + 2 * NB:2 + 3 * NB])
    ss = list(scr[2 + 3 * NB:2 + 4 * NB])
    fs = list(scr[2 + 4 * NB:2 + 5 * NB])
    chunks = EW // cc

    c = lax.axis_index("c")
    s = lax.axis_index("s")
    wid = c * NS + s
    rbase = pl.multiple_of(s * RPT, 8)
    ebase = pl.multiple_of(wid * EW, 8)

    pltpu.sync_copy(src_hbm.at[pl.ds(ebase, EW)], sidx_v)

    def zrow(i, carry):
        for j in range(d // 16):
            rows[0][i, pl.ds(j * 16, 16)] = jnp.zeros((16,), jnp.float32)
        return carry
    lax.fori_loop(0, cc, zrow, 0)

    def zcopy(i, carry):
        pltpu.sync_copy(rows[0], acc_sh.at[pl.ds(rbase + i * cc, cc)])
        return carry
    lax.fori_loop(0, RPT // cc, zcopy, 0)
    if RPT % cc:
        pltpu.sync_copy(rows[0].at[pl.ds(0, RPT % cc)],
                        acc_sh.at[pl.ds(rbase + (RPT // cc) * cc, RPT % cc)])
    plsc.subcore_barrier()

    def start_gather(i, b):
        pltpu.async_copy(hs_hbm.at[sidx_v.at[pl.ds(i * cc, cc)]], rows[b],
                         gs[b])

    def wait_gather(i, b):
        pltpu.make_async_copy(hs_hbm.at[sidx_v.at[pl.ds(i * cc, cc)]],
                              rows[b], gs[b]).wait()

    def start_didx(i, b):
        pltpu.async_copy(dst_hbm.at[pl.ds(ebase + i * cc, cc)], didx[b],
                         fs[b])

    def wait_didx(i, b):
        pltpu.make_async_copy(dst_hbm.at[pl.ds(ebase + i * cc, cc)], didx[b],
                              fs[b]).wait()

    def start_scat(b):
        pltpu.async_copy(rows[b], acc_sh.at[didx[b]], ss[b], add=True)

    def wait_scat(b):
        pltpu.make_async_copy(rows[b], acc_sh.at[didx[b]], ss[b]).wait()

    for b in range(NB):
        start_didx(b, b)
        start_gather(b, b)

    def ring(j, carry):
        i0 = j * NB
        for b in range(NB):
            wait_gather(i0 + b, b)
            wait_didx(i0 + b, b)
            start_scat(b)
        for b in range(NB):
            wait_scat(b)
            start_gather(i0 + NB + b, b)
            start_didx(i0 + NB + b, b)
        return carry
    lax.fori_loop(0, chunks // NB - 1, ring, 0)
    last = chunks - NB
    for b in range(NB):
        wait_gather(last + b, b)
        wait_didx(last + b, b)
        start_scat(b)
    for b in range(NB):
        wait_scat(b)
    plsc.subcore_barrier()

    def ocopy(i, carry):
        ob = pl.multiple_of(rbase + i * cc, 8)
        pltpu.sync_copy(acc_sh.at[pl.ds(ob, cc)], rows[1])
        pltpu.sync_copy(rows[1], out_hbm.at[c, pl.ds(ob, cc)])
        return carry
    lax.fori_loop(0, RPT // cc, ocopy, 0)
    if RPT % cc:
        ot = pl.multiple_of(rbase + (RPT // cc) * cc, 8)
        pltpu.sync_copy(acc_sh.at[pl.ds(ot, RPT % cc)],
                        rows[1].at[pl.ds(0, RPT % cc)])
        pltpu.sync_copy(rows[1].at[pl.ds(0, RPT % cc)],
                        out_hbm.at[c, pl.ds(ot, RPT % cc)])


def _make_agg(d, cc, NB, tc_tiling=True):
    return pl.kernel(
        functools.partial(_aggn_body, d=d, cc=cc, NB=NB),
        out_type=jax.ShapeDtypeStruct((NC, NACC, d), jnp.float32),
        mesh=_MESH,
        compiler_params=pltpu.CompilerParams(use_tc_tiling_on_sc=tc_tiling),
        scratch_types=(
            [pltpu.VMEM((EW,), jnp.int32)]
            + [pltpu.VMEM((cc,), jnp.int32) for _ in range(NB)]
            + [pltpu.VMEM((cc, d), jnp.float32) for _ in range(NB)]
            + [pltpu.VMEM_SHARED((NACC, d), jnp.float32)]
            + [pltpu.SemaphoreType.DMA for _ in range(3 * NB)]
        ),
    )


_agg128 = _make_agg(HID, 32, 8)
_agg64 = _make_agg(OUT, C, 8, tc_tiling=False)

BM = 400


def _m1_body(x_ref, w_ref, dv_ref, hs_ref):
    h = jnp.dot(x_ref[...], w_ref[...], preferred_element_type=jnp.float32)
    hs_ref[...] = h * dv_ref[...]


_m1_call = pl.pallas_call(
    _m1_body,
    grid=(N // BM,),
    in_specs=[
        pl.BlockSpec((BM, F_IN), lambda i: (i, 0)),
        pl.BlockSpec((F_IN, HID), lambda i: (0, 0)),
        pl.BlockSpec((BM, HID), lambda i: (i, 0)),
    ],
    out_specs=pl.BlockSpec((BM, HID), lambda i: (i, 0)),
    out_shape=jax.ShapeDtypeStruct((N, HID), jnp.float32),
)


def _l2_body(a0_ref, a1_ref, hs1_ref, dv_ref, b1_ref, w2_ref, o_ref):
    act = (a0_ref[0] + a1_ref[0] + hs1_ref[...]) * dv_ref[...] + b1_ref[...]
    act = jnp.maximum(act, 0.0)
    h2 = jnp.dot(act, w2_ref[...], preferred_element_type=jnp.float32)
    o_ref[...] = h2 * dv_ref[:, :OUT]


_l2_call = pl.pallas_call(
    _l2_body,
    grid=(N // BM,),
    in_specs=[
        pl.BlockSpec((1, BM, HID), lambda i: (0, i, 0)),
        pl.BlockSpec((1, BM, HID), lambda i: (1, i, 0)),
        pl.BlockSpec((BM, HID), lambda i: (i, 0)),
        pl.BlockSpec((BM, HID), lambda i: (i, 0)),
        pl.BlockSpec((1, HID), lambda i: (0, 0)),
        pl.BlockSpec((HID, OUT), lambda i: (0, 0)),
    ],
    out_specs=pl.BlockSpec((BM, OUT), lambda i: (i, 0)),
    out_shape=jax.ShapeDtypeStruct((N, OUT), jnp.float32),
)


def _fin_body(a0_ref, a1_ref, hs2_ref, dv_ref, b2_ref, o_ref):
    t = a0_ref[0] + a1_ref[0] + hs2_ref[...]
    o = t * dv_ref[:, :OUT] + b2_ref[...]
    o_ref[...] = jax.nn.sigmoid(o)


_fin_call = pl.pallas_call(
    _fin_body,
    grid=(N // BM,),
    in_specs=[
        pl.BlockSpec((1, BM, OUT), lambda i: (0, i, 0)),
        pl.BlockSpec((1, BM, OUT), lambda i: (1, i, 0)),
        pl.BlockSpec((BM, OUT), lambda i: (i, 0)),
        pl.BlockSpec((BM, HID), lambda i: (i, 0)),
        pl.BlockSpec((1, OUT), lambda i: (0, 0)),
    ],
    out_specs=pl.BlockSpec((BM, OUT), lambda i: (i, 0)),
    out_shape=jax.ShapeDtypeStruct((N, OUT), jnp.float32),
)


def kernel(x, edge_index, W1, b1, W2, b2):
    src = edge_index[0]
    dst = edge_index[1]
    pad_i = jnp.arange(PAD, dtype=jnp.int32)
    src_p = jnp.concatenate([src, (pad_i * 131) % N])
    dst_f = jnp.concatenate([dst, N + (pad_i % TRASH)])
    dst_p = dst_f.reshape(-1, C)

    degf = _deg_call(dst_p)
    deg = degf[:N] + degf[NACC:NACC + N] + 1.0
    dinv = jnp.broadcast_to(lax.rsqrt(deg)[:, None], (N, HID))

    hs1 = _m1_call(x, W1, dinv)
    agg1 = _agg128(hs1, src_p, dst_f)
    hs2 = _l2_call(agg1, agg1, hs1, dinv, b1[None, :], W2)
    agg2 = _agg64(hs2, src_p, dst_f)
    return _fin_call(agg2, agg2, hs2, dinv, b2[None, :])
